# baseline (speedup 1.0000x reference)
import functools

import jax
import jax.numpy as jnp
from jax import lax
from jax.experimental import pallas as pl
from jax.experimental.pallas import tpu as pltpu
from jax.experimental.pallas import tpu_sc as plsc

NC = 2
NS = 16
NW = NC * NS
LANES = 16


@functools.lru_cache(maxsize=None)
def _make_sc_scores(B, K, D):
    KP1 = K + 1
    EPW = B // NW
    C = 16
    NCHUNK = EPW // C
    NG = C // LANES
    ROWS_PER_CHUNK = C * (K + 2)

    mesh = plsc.VectorSubcoreMesh(core_axis_name="c", subcore_axis_name="s")

    @functools.partial(
        pl.kernel,
        mesh=mesh,
        compiler_params=pltpu.CompilerParams(needs_layout_passes=False,
                                             use_tc_tiling_on_sc=True),
        out_type=jax.ShapeDtypeStruct((NW, KP1, EPW), jnp.float32),
        scratch_types=[
            pltpu.VMEM((EPW,), jnp.int32),
            pltpu.VMEM((EPW,), jnp.int32),
            pltpu.VMEM((K, EPW), jnp.int32),
            pltpu.VMEM((2 * C, D), jnp.float32),
            pltpu.VMEM((2 * C, D), jnp.float32),
            pltpu.VMEM((2 * C * K, D), jnp.float32),
            pltpu.VMEM((KP1, EPW), jnp.float32),
            pltpu.SemaphoreType.DMA,
            pltpu.SemaphoreType.DMA,
        ],
    )
    def sc_scores(emb2_hbm, ctx2_hbm, tgt_hbm, ctxi_hbm, negt_hbm, out_hbm,
                  tgt_v, ctxi_v, negt_v, vw_v, vc_v, vn_v, sc_v,
                  semA, semB):
        wid = lax.axis_index("s") * NC + lax.axis_index("c")
        base = wid * EPW
        iota = lax.iota(jnp.int32, LANES)
        pltpu.sync_copy(tgt_hbm.at[pl.ds(base, EPW)], tgt_v)
        pltpu.sync_copy(ctxi_hbm.at[pl.ds(base, EPW)], ctxi_v)
        pltpu.sync_copy(negt_hbm.at[pl.ds(0, K), pl.ds(base, EPW)], negt_v)

        def fire(c, p, sem):
            s0 = pl.multiple_of(c * C, C)
            for g in range(NG):
                vt = tgt_v[pl.ds(s0 + g * LANES, LANES)]
                vx = ctxi_v[pl.ds(s0 + g * LANES, LANES)]
                for lane in range(LANES):
                    dst = p * C + g * LANES + lane
                    vtl = vt[lane]
                    pltpu.make_async_copy(
                        emb2_hbm.at[vtl >> 1, pl.ds((vtl & 1) * D, D)],
                        vw_v.at[dst], sem).start()
                    vxl = vx[lane]
                    pltpu.make_async_copy(
                        ctx2_hbm.at[vxl >> 1, pl.ds((vxl & 1) * D, D)],
                        vc_v.at[dst], sem).start()

            def issue_neg(k, carry):
                for g in range(NG):
                    vn = negt_v[k, pl.ds(s0 + g * LANES, LANES)]
                    for lane in range(LANES):
                        dst = p * C * K + g * LANES + lane + k * C
                        vnl = vn[lane]
                        pltpu.make_async_copy(
                            ctx2_hbm.at[vnl >> 1, pl.ds((vnl & 1) * D, D)],
                            vn_v.at[dst], sem).start()
                return carry

            lax.fori_loop(0, K, issue_neg, 0)

        def drain(sem):
            def wait_one(i, carry):
                pltpu.make_async_copy(
                    emb2_hbm.at[0, pl.ds(0, D)], vw_v.at[0], sem).wait()
                return carry

            lax.fori_loop(0, ROWS_PER_CHUNK, wait_one, 0)

        def compute(c, p):
            for g in range(NG):
                rows = p * C + g * LANES + iota
                nbase = p * C * K + g * LANES + iota

                def body(d, accs):
                    dcol = jnp.full((LANES,), d, jnp.int32)
                    vw_d = plsc.load_gather(vw_v, [rows, dcol])
                    vc_d = plsc.load_gather(vc_v, [rows, dcol])
                    new = [accs[0] + vw_d * vc_d]
                    for k in range(K):
                        vn_d = plsc.load_gather(vn_v, [nbase + k * C, dcol])
                        new.append(accs[k + 1] + vn_d * vw_d)
                    return tuple(new)

                accs = lax.fori_loop(
                    0, D, body,
                    tuple(jnp.zeros((LANES,), jnp.float32) for _ in range(KP1)))
                off = c * C + g * LANES
                for k in range(KP1):
                    sc_v[k, pl.ds(off, LANES)] = accs[k]

        fire(0, 0, semA)
        fire(1, 1, semB)

        def pair_body(j, carry):
            c0 = 2 * j
            drain(semA)
            compute(c0, 0)
            fire(c0 + 2, 0, semA)
            c1 = c0 + 1
            drain(semB)
            compute(c1, 1)
            fire(c1 + 2, 1, semB)
            return carry

        lax.fori_loop(0, NCHUNK // 2 - 1, pair_body, 0)
        cl = NCHUNK - 2
        drain(semA)
        compute(cl, 0)
        drain(semB)
        compute(cl + 1, 1)
        pltpu.sync_copy(sc_v, out_hbm.at[wid])

    return sc_scores


@functools.lru_cache(maxsize=None)
def _make_loss(B, K):
    KP1 = K + 1
    EPW = B // NW

    def loss_body(s_ref, o_ref):
        x = s_ref[...]
        r = lax.broadcasted_iota(jnp.int32, x.shape, 1)
        t = jnp.where(r == 0, -x, x)
        sp = jnp.maximum(t, 0.0) + jnp.log(1.0 + jnp.exp(-jnp.abs(t)))
        o_ref[0, 0] = jnp.sum(sp) / B

    def loss(scores):
        out = pl.pallas_call(
            loss_body,
            out_shape=jax.ShapeDtypeStruct((1, 1), jnp.float32),
            out_specs=pl.BlockSpec(memory_space=pltpu.SMEM),
        )(scores)
        return out[0, 0]

    return loss


def kernel(target, context, negatives, emb, ctx_emb):
    B, = target.shape
    _, K = negatives.shape
    _, D = emb.shape
    V = emb.shape[0]
    tgt = target.astype(jnp.int32)
    ctxi = context.astype(jnp.int32)
    negt = negatives.astype(jnp.int32).T
    emb2 = emb.reshape(V // 2, 2 * D)
    ctx2 = ctx_emb.reshape(V // 2, 2 * D)
    scores = _make_sc_scores(B, K, D)(emb2, ctx2, tgt, ctxi, negt)
    return _make_loss(B, K)(scores)


# speedup vs baseline: 1.3171x; 1.3171x over previous
import functools

import jax
import jax.numpy as jnp
from jax import lax
from jax.experimental import pallas as pl
from jax.experimental.pallas import tpu as pltpu
from jax.experimental.pallas import tpu_sc as plsc

NC = 2
NS = 16
NW = NC * NS
LANES = 16


@functools.lru_cache(maxsize=None)
def _make_sc_scores(B, K, D):
    KP1 = K + 1
    EPW = B // NW
    C = 16
    NCHUNK = EPW // C
    NG = C // LANES
    ROWS_PER_CHUNK = C * (K + 2)

    mesh = plsc.VectorSubcoreMesh(core_axis_name="c", subcore_axis_name="s")

    @functools.partial(
        pl.kernel,
        mesh=mesh,
        compiler_params=pltpu.CompilerParams(needs_layout_passes=False,
                                             use_tc_tiling_on_sc=True),
        out_type=jax.ShapeDtypeStruct((NW, KP1, EPW), jnp.float32),
        scratch_types=[
            pltpu.VMEM((EPW,), jnp.int32),
            pltpu.VMEM((EPW,), jnp.int32),
            pltpu.VMEM((K, EPW), jnp.int32),
            pltpu.VMEM((2 * C, D), jnp.float32),
            pltpu.VMEM((2 * C, D), jnp.float32),
            pltpu.VMEM((2 * C * K, D), jnp.float32),
            pltpu.VMEM((KP1, EPW), jnp.float32),
            pltpu.SemaphoreType.DMA,
            pltpu.SemaphoreType.DMA,
        ],
    )
    def sc_scores(emb_hbm, ctx_hbm, tgt_hbm, ctxi_hbm, negt_hbm, out_hbm,
                  tgt_v, ctxi_v, negt_v, vw_v, vc_v, vn_v, sc_v,
                  semA, semB):
        wid = lax.axis_index("s") * NC + lax.axis_index("c")
        base = wid * EPW
        iota = lax.iota(jnp.int32, LANES)
        pltpu.sync_copy(tgt_hbm.at[pl.ds(base, EPW)], tgt_v)
        pltpu.sync_copy(ctxi_hbm.at[pl.ds(base, EPW)], ctxi_v)
        pltpu.sync_copy(negt_hbm.at[pl.ds(0, K), pl.ds(base, EPW)], negt_v)

        def fire(c, p, sem):
            s0 = pl.multiple_of(c * C, C)
            for g in range(NG):
                vt = tgt_v[pl.ds(s0 + g * LANES, LANES)]
                vx = ctxi_v[pl.ds(s0 + g * LANES, LANES)]
                for lane in range(LANES):
                    dst = p * C + g * LANES + lane
                    pltpu.make_async_copy(
                        emb_hbm.at[vt[lane]], vw_v.at[dst], sem).start()
                    pltpu.make_async_copy(
                        ctx_hbm.at[vx[lane]], vc_v.at[dst], sem).start()

            def issue_neg(k, carry):
                for g in range(NG):
                    vn = negt_v[k, pl.ds(s0 + g * LANES, LANES)]
                    for lane in range(LANES):
                        dst = p * C * K + g * LANES + lane + k * C
                        pltpu.make_async_copy(
                            ctx_hbm.at[vn[lane]], vn_v.at[dst], sem).start()
                return carry

            lax.fori_loop(0, K, issue_neg, 0)

        def drain(sem):
            def wait_one(i, carry):
                pltpu.make_async_copy(
                    emb_hbm.at[0], vw_v.at[0], sem).wait()
                return carry

            lax.fori_loop(0, ROWS_PER_CHUNK, wait_one, 0)

        def compute(c, p):
            for g in range(NG):
                rows = p * C + g * LANES + iota
                nbase = p * C * K + g * LANES + iota

                def body(d, accs):
                    dcol = jnp.full((LANES,), d, jnp.int32)
                    vw_d = plsc.load_gather(vw_v, [rows, dcol])
                    vc_d = plsc.load_gather(vc_v, [rows, dcol])
                    new = [accs[0] + vw_d * vc_d]
                    for k in range(K):
                        vn_d = plsc.load_gather(vn_v, [nbase + k * C, dcol])
                        new.append(accs[k + 1] + vn_d * vw_d)
                    return tuple(new)

                accs = lax.fori_loop(
                    0, D, body,
                    tuple(jnp.zeros((LANES,), jnp.float32) for _ in range(KP1)))
                off = c * C + g * LANES
                for k in range(KP1):
                    sc_v[k, pl.ds(off, LANES)] = accs[k]

        fire(0, 0, semA)
        fire(1, 1, semB)

        def pair_body(j, carry):
            c0 = 2 * j
            drain(semA)
            compute(c0, 0)
            fire(c0 + 2, 0, semA)
            c1 = c0 + 1
            drain(semB)
            compute(c1, 1)
            fire(c1 + 2, 1, semB)
            return carry

        lax.fori_loop(0, NCHUNK // 2 - 1, pair_body, 0)
        cl = NCHUNK - 2
        drain(semA)
        compute(cl, 0)
        drain(semB)
        compute(cl + 1, 1)
        pltpu.sync_copy(sc_v, out_hbm.at[wid])

    return sc_scores


@functools.lru_cache(maxsize=None)
def _make_loss(B, K):
    KP1 = K + 1
    EPW = B // NW

    def loss_body(s_ref, o_ref):
        x = s_ref[...]
        r = lax.broadcasted_iota(jnp.int32, x.shape, 1)
        t = jnp.where(r == 0, -x, x)
        sp = jnp.maximum(t, 0.0) + jnp.log(1.0 + jnp.exp(-jnp.abs(t)))
        o_ref[0, 0] = jnp.sum(sp) / B

    def loss(scores):
        out = pl.pallas_call(
            loss_body,
            out_shape=jax.ShapeDtypeStruct((1, 1), jnp.float32),
            out_specs=pl.BlockSpec(memory_space=pltpu.SMEM),
        )(scores)
        return out[0, 0]

    return loss


def kernel(target, context, negatives, emb, ctx_emb):
    B, = target.shape
    _, K = negatives.shape
    _, D = emb.shape
    tgt = target.astype(jnp.int32)
    ctxi = context.astype(jnp.int32)
    negt = negatives.astype(jnp.int32).T
    scores = _make_sc_scores(B, K, D)(emb, ctx_emb, tgt, ctxi, negt)
    return _make_loss(B, K)(scores)
